# baseline (speedup 1.0000x reference)
import jax
import jax.numpy as jnp
from jax.experimental import pallas as pl
from jax.experimental.pallas import tpu as pltpu

N_NODE = 128
H = 256
RH = 128
T = 128
C = 128

_F32 = jnp.float32


def _dot(a, b):
    return jax.lax.dot_general(a, b, (((1,), (0,)), ((), ())),
                               preferred_element_type=_F32)


def _gnn_kernel(cfeats_ref, ctypes_ref, op_emb_ref, c_ei_ref, c_ef_ref,
                tfeats_ref, t_ei_ref, t_ef_ref,
                *rest):
    c_wx = rest[0:6]
    c_we = rest[6:12]
    c_b_ref = rest[12]
    t_wx = rest[13:17]
    t_we = rest[17:21]
    t_b_ref = rest[21]
    c_out_ref, t_out_ref = rest[22], rest[23]

    def run_graph(x0, ei_ref, ef_ref, wx_refs, we_refs, b_ref, acts, resids):
        n_e = ei_ref.shape[1]
        src = ei_ref[0:1, :]
        dst = ei_ref[1:2, :]
        e_iota = jax.lax.broadcasted_iota(jnp.int32, (N_NODE, n_e), 0)
        src1h = (e_iota == src).astype(_F32)
        dst1h = (e_iota == dst).astype(_F32)
        adj = jax.lax.dot_general(dst1h, src1h, (((1,), (1,)), ((), ())),
                                  preferred_element_type=_F32)
        eagg = _dot(dst1h, ef_ref[...])
        ones = jnp.ones((N_NODE, 1), _F32)
        in_deg = _dot(adj, ones)
        out_deg = jax.lax.dot_general(adj, ones, (((0,), (0,)), ((), ())),
                                      preferred_element_type=_F32)
        ood = jax.lax.rsqrt(jnp.maximum(out_deg, 1.0))
        ind = jax.lax.rsqrt(jnp.maximum(in_deg, 1.0))

        x = x0
        for i in range(len(wx_refs)):
            feat = x * ood
            agg = _dot(adj, feat)
            rst = _dot(agg, wx_refs[i][...]) + _dot(eagg, we_refs[i][...])
            rst = rst * ind + b_ref[i:i + 1, :]
            if acts[i]:
                rst = jax.nn.sigmoid(rst)
            if resids[i]:
                rst = feat + rst
            x = rst
        return x

    n_ops = op_emb_ref.shape[0]
    t_iota = jax.lax.broadcasted_iota(jnp.int32, (n_ops, N_NODE), 0)
    op1h_t = (t_iota == ctypes_ref[0:1, :]).astype(_F32)
    op = jax.lax.dot_general(op1h_t, op_emb_ref[...],
                             (((0,), (0,)), ((), ())),
                             preferred_element_type=_F32)
    x0c = jnp.concatenate([cfeats_ref[...], op], axis=1)

    c_emb = run_graph(x0c, c_ei_ref, c_ef_ref, c_wx, c_we, c_b_ref,
                      acts=[True, True, True, True, True, False],
                      resids=[False, True, True, True, True, False])
    c_out_ref[...] = c_emb

    t_emb = run_graph(tfeats_ref[...], t_ei_ref, t_ef_ref, t_wx, t_we,
                      t_b_ref,
                      acts=[True, True, True, False],
                      resids=[False, True, True, False])
    t_out_ref[...] = t_emb


def _gru_step(h, gx, gh):
    z = jax.nn.sigmoid(gx[:, 0:RH] + gh[:, 0:RH])
    r = jax.nn.sigmoid(gx[:, RH:2 * RH] + gh[:, RH:2 * RH])
    hc = jnp.tanh(gx[:, 2 * RH:] + r * gh[:, 2 * RH:])
    return z * h + (1.0 - z) * hc


def _rnn_kernel(c_emb_ref, t_emb_ref,
                kc_f_ref, kt_f_ref, r1_f_ref, b1_f_ref,
                kc_b_ref, kt_b_ref, r1_b_ref, b1_b_ref,
                k2_f_ref, r2_f_ref, b2_f_ref,
                k2_b_ref, r2_b_ref, b2_b_ref,
                wf_ref, wb_ref, fb_ref,
                out_ref, y1_ref, gxt_f_ref, gxt_b_ref):
    c_emb = c_emb_ref[...]
    t_emb = t_emb_ref[...]
    gxc_f = _dot(c_emb, kc_f_ref[...]) + b1_f_ref[0:1, :]
    gxt_f_ref[...] = _dot(t_emb, kt_f_ref[...])
    gxc_b = _dot(c_emb, kc_b_ref[...]) + b1_b_ref[0:1, :]
    gxt_b_ref[...] = _dot(t_emb, kt_b_ref[...])

    cb = c_emb.shape[0]
    r1_f = r1_f_ref[...]
    r1_b = r1_b_ref[...]
    bh1_f = b1_f_ref[1:2, :]
    bh1_b = b1_b_ref[1:2, :]

    def loop1(k, carry):
        h_f, h_b = carry
        tb = T - 1 - k
        gx_f = gxc_f + gxt_f_ref[pl.ds(k, 1), :]
        gx_b = gxc_b + gxt_b_ref[pl.ds(tb, 1), :]
        gh_f = _dot(h_f, r1_f) + bh1_f
        gh_b = _dot(h_b, r1_b) + bh1_b
        h_f = _gru_step(h_f, gx_f, gh_f)
        h_b = _gru_step(h_b, gx_b, gh_b)
        y1_ref[k, :, 0:RH] = h_f
        y1_ref[tb, :, RH:2 * RH] = h_b
        return h_f, h_b

    h0 = jnp.zeros((cb, RH), _F32)
    jax.lax.fori_loop(0, T, loop1, (h0, h0))

    k2_f = k2_f_ref[...]
    r2_f = r2_f_ref[...]
    k2_b = k2_b_ref[...]
    r2_b = r2_b_ref[...]
    bx2_f = b2_f_ref[0:1, :]
    bh2_f = b2_f_ref[1:2, :]
    bx2_b = b2_b_ref[0:1, :]
    bh2_b = b2_b_ref[1:2, :]
    wf = wf_ref[...]
    wb = wb_ref[...]

    out_ref[...] = jnp.broadcast_to(fb_ref[...].reshape(1, 1, 2), (T, cb, 2))

    def loop2(k, carry):
        h_f, h_b = carry
        tb = T - 1 - k
        x_f = y1_ref[k]
        x_b = y1_ref[tb]
        gx_f = _dot(x_f, k2_f) + bx2_f
        gx_b = _dot(x_b, k2_b) + bx2_b
        gh_f = _dot(h_f, r2_f) + bh2_f
        gh_b = _dot(h_b, r2_b) + bh2_b
        h_f = _gru_step(h_f, gx_f, gh_f)
        h_b = _gru_step(h_b, gx_b, gh_b)
        out_ref[k] = out_ref[k] + _dot(h_f, wf)
        out_ref[tb] = out_ref[tb] + _dot(h_b, wb)
        return h_f, h_b

    jax.lax.fori_loop(0, T, loop2, (h0, h0))

    logits = out_ref[...]
    a = logits[:, :, 0:1]
    b = logits[:, :, 1:2]
    m = jnp.maximum(a, b)
    lse = m + jnp.log(jnp.exp(a - m) + jnp.exp(b - m))
    out_ref[...] = logits - lse


def kernel(cfeats, cedge_feats, ctypes, tfeats, tedge_feats, combined_feats,
           cgraph_edge_index, tgraph_edge_index, params):
    del combined_feats
    op_emb = params['op_emb']
    c_w, c_b = params['c_W'], params['c_b']
    t_w, t_b = params['t_W'], params['t_b']

    c_wx = [w[:-16, :] for w in c_w]
    c_we = [w[-16:, :] for w in c_w]
    t_wx = [w[:-16, :] for w in t_w]
    t_we = [w[-16:, :] for w in t_w]
    c_b_stack = jnp.stack(c_b, axis=0)
    t_b_stack = jnp.stack(t_b, axis=0)

    gnn_in = ([cfeats, ctypes.astype(jnp.int32).reshape(1, N_NODE), op_emb,
               cgraph_edge_index.astype(jnp.int32),
               cedge_feats, tfeats,
               tgraph_edge_index.astype(jnp.int32), tedge_feats]
              + c_wx + c_we + [c_b_stack] + t_wx + t_we + [t_b_stack])

    c_emb, t_emb = pl.pallas_call(
        _gnn_kernel,
        out_shape=[jax.ShapeDtypeStruct((N_NODE, H), _F32),
                   jax.ShapeDtypeStruct((N_NODE, H), _F32)],
    )(*gnn_in)

    k1_f, r1_f, b1_f = params['gru1_f']
    k1_b, r1_b, b1_b = params['gru1_b']
    k2_f, r2_f, b2_f = params['gru2_f']
    k2_b, r2_b, b2_b = params['gru2_b']
    fw = params['final_W']
    fb = params['final_b']

    nsplit = 2
    cb = C // nsplit
    full2d = lambda shape: pl.BlockSpec(shape, lambda i: (0, 0))
    out_tc = pl.pallas_call(
        _rnn_kernel,
        grid=(nsplit,),
        in_specs=[pl.BlockSpec((cb, H), lambda i: (i, 0)),
                  full2d((N_NODE, H)),
                  full2d((H, 3 * RH)), full2d((H, 3 * RH)),
                  full2d((RH, 3 * RH)), full2d((2, 3 * RH)),
                  full2d((H, 3 * RH)), full2d((H, 3 * RH)),
                  full2d((RH, 3 * RH)), full2d((2, 3 * RH)),
                  full2d((2 * RH, 3 * RH)), full2d((RH, 3 * RH)),
                  full2d((2, 3 * RH)),
                  full2d((2 * RH, 3 * RH)), full2d((RH, 3 * RH)),
                  full2d((2, 3 * RH)),
                  full2d((RH, 2)), full2d((RH, 2)), full2d((1, 2))],
        out_specs=pl.BlockSpec((T, cb, 2), lambda i: (0, i, 0)),
        out_shape=jax.ShapeDtypeStruct((T, C, 2), _F32),
        scratch_shapes=[pltpu.VMEM((T, cb, 2 * RH), _F32),
                        pltpu.VMEM((T, 3 * RH), _F32),
                        pltpu.VMEM((T, 3 * RH), _F32)],
        compiler_params=pltpu.CompilerParams(
            dimension_semantics=("parallel",)),
    )(c_emb, t_emb,
      k1_f[:H, :], k1_f[H:, :], r1_f, b1_f,
      k1_b[:H, :], k1_b[H:, :], r1_b, b1_b,
      k2_f, r2_f, b2_f,
      k2_b, r2_b, b2_b,
      fw[:RH, :], fw[RH:, :], fb.reshape(1, 2))

    return jnp.swapaxes(out_tc, 0, 1)


# speedup vs baseline: 1.3531x; 1.3531x over previous
import jax
import jax.numpy as jnp
from jax.experimental import pallas as pl
from jax.experimental.pallas import tpu as pltpu

N_NODE = 128
H = 256
RH = 128
T = 128
C = 128

_F32 = jnp.float32


_BF16 = jnp.bfloat16


def _dot(a, b):
    return jax.lax.dot_general(a, b, (((1,), (0,)), ((), ())),
                               preferred_element_type=_F32)


def _dotb(a, b):
    return jax.lax.dot_general(a.astype(_BF16), b, (((1,), (0,)), ((), ())),
                               preferred_element_type=_F32)


def _gnn_kernel(cfeats_ref, ctypes_ref, op_emb_ref, c_ei_ref, c_ef_ref,
                tfeats_ref, t_ei_ref, t_ef_ref,
                *rest):
    c_wx = rest[0:6]
    c_we = rest[6:12]
    c_b_ref = rest[12]
    t_wx = rest[13:17]
    t_we = rest[17:21]
    t_b_ref = rest[21]
    c_out_ref, t_out_ref = rest[22], rest[23]

    def run_graph(x0, ei_ref, ef_ref, wx_refs, we_refs, b_ref, acts, resids):
        n_e = ei_ref.shape[1]
        src = ei_ref[0:1, :]
        dst = ei_ref[1:2, :]
        e_iota = jax.lax.broadcasted_iota(jnp.int32, (N_NODE, n_e), 0)
        src1h = (e_iota == src).astype(_F32)
        dst1h = (e_iota == dst).astype(_F32)
        adj = jax.lax.dot_general(dst1h, src1h, (((1,), (1,)), ((), ())),
                                  preferred_element_type=_F32)
        eagg = _dot(dst1h, ef_ref[...])
        ones = jnp.ones((N_NODE, 1), _F32)
        in_deg = _dot(adj, ones)
        out_deg = jax.lax.dot_general(adj, ones, (((0,), (0,)), ((), ())),
                                      preferred_element_type=_F32)
        ood = jax.lax.rsqrt(jnp.maximum(out_deg, 1.0))
        ind = jax.lax.rsqrt(jnp.maximum(in_deg, 1.0))

        x = x0
        for i in range(len(wx_refs)):
            feat = x * ood
            agg = _dot(adj, feat)
            rst = _dot(agg, wx_refs[i][...]) + _dot(eagg, we_refs[i][...])
            rst = rst * ind + b_ref[i:i + 1, :]
            if acts[i]:
                rst = jax.nn.sigmoid(rst)
            if resids[i]:
                rst = feat + rst
            x = rst
        return x

    n_ops = op_emb_ref.shape[0]
    t_iota = jax.lax.broadcasted_iota(jnp.int32, (n_ops, N_NODE), 0)
    op1h_t = (t_iota == ctypes_ref[0:1, :]).astype(_F32)
    op = jax.lax.dot_general(op1h_t, op_emb_ref[...],
                             (((0,), (0,)), ((), ())),
                             preferred_element_type=_F32)
    x0c = jnp.concatenate([cfeats_ref[...], op], axis=1)

    c_emb = run_graph(x0c, c_ei_ref, c_ef_ref, c_wx, c_we, c_b_ref,
                      acts=[True, True, True, True, True, False],
                      resids=[False, True, True, True, True, False])
    c_out_ref[...] = c_emb

    t_emb = run_graph(tfeats_ref[...], t_ei_ref, t_ef_ref, t_wx, t_we,
                      t_b_ref,
                      acts=[True, True, True, False],
                      resids=[False, True, True, False])
    t_out_ref[...] = t_emb


def _gru_step(h, gx, gh):
    z = jax.nn.sigmoid(gx[:, 0:RH] + gh[:, 0:RH])
    r = jax.nn.sigmoid(gx[:, RH:2 * RH] + gh[:, RH:2 * RH])
    hc = jnp.tanh(gx[:, 2 * RH:] + r * gh[:, 2 * RH:])
    return z * h + (1.0 - z) * hc


def _rnn_kernel(c_emb_ref, t_emb_ref,
                kc_f_ref, kt_f_ref, r1_f_ref, b1_f_ref,
                kc_b_ref, kt_b_ref, r1_b_ref, b1_b_ref,
                k2_f_ref, r2_f_ref, b2_f_ref,
                k2_b_ref, r2_b_ref, b2_b_ref,
                wf_ref, wb_ref, fb_ref,
                out_ref, y1_ref, gxt_f_ref, gxt_b_ref):
    c_emb = c_emb_ref[...]
    t_emb = t_emb_ref[...]
    gxc_f = _dot(c_emb, kc_f_ref[...]) + b1_f_ref[0:1, :]
    gxt_f_ref[...] = _dot(t_emb, kt_f_ref[...])
    gxc_b = _dot(c_emb, kc_b_ref[...]) + b1_b_ref[0:1, :]
    gxt_b_ref[...] = _dot(t_emb, kt_b_ref[...])

    cb = c_emb.shape[0]
    r1_f = r1_f_ref[...].astype(_BF16)
    r1_b = r1_b_ref[...].astype(_BF16)
    bh1_f = b1_f_ref[1:2, :]
    bh1_b = b1_b_ref[1:2, :]

    def loop1(k, carry):
        h_f, h_b = carry
        tb = T - 1 - k
        gx_f = gxc_f + gxt_f_ref[pl.ds(k, 1), :]
        gx_b = gxc_b + gxt_b_ref[pl.ds(tb, 1), :]
        gh_f = _dotb(h_f, r1_f) + bh1_f
        gh_b = _dotb(h_b, r1_b) + bh1_b
        h_f = _gru_step(h_f, gx_f, gh_f)
        h_b = _gru_step(h_b, gx_b, gh_b)
        y1_ref[k, :, 0:RH] = h_f
        y1_ref[tb, :, RH:2 * RH] = h_b
        return h_f, h_b

    h0 = jnp.zeros((cb, RH), _F32)
    jax.lax.fori_loop(0, T, loop1, (h0, h0))

    k2_f = k2_f_ref[...].astype(_BF16)
    r2_f = r2_f_ref[...].astype(_BF16)
    k2_b = k2_b_ref[...].astype(_BF16)
    r2_b = r2_b_ref[...].astype(_BF16)
    bx2_f = b2_f_ref[0:1, :]
    bh2_f = b2_f_ref[1:2, :]
    bx2_b = b2_b_ref[0:1, :]
    bh2_b = b2_b_ref[1:2, :]
    wf = wf_ref[...]
    wb = wb_ref[...]

    out_ref[...] = jnp.broadcast_to(fb_ref[...].reshape(1, 1, 2), (T, cb, 2))

    def loop2(k, carry):
        h_f, h_b = carry
        tb = T - 1 - k
        x_f = y1_ref[k]
        x_b = y1_ref[tb]
        gx_f = _dotb(x_f, k2_f) + bx2_f
        gx_b = _dotb(x_b, k2_b) + bx2_b
        gh_f = _dotb(h_f, r2_f) + bh2_f
        gh_b = _dotb(h_b, r2_b) + bh2_b
        h_f = _gru_step(h_f, gx_f, gh_f)
        h_b = _gru_step(h_b, gx_b, gh_b)
        out_ref[k] = out_ref[k] + _dot(h_f, wf)
        out_ref[tb] = out_ref[tb] + _dot(h_b, wb)
        return h_f, h_b

    jax.lax.fori_loop(0, T, loop2, (h0, h0))

    logits = out_ref[...]
    a = logits[:, :, 0:1]
    b = logits[:, :, 1:2]
    m = jnp.maximum(a, b)
    lse = m + jnp.log(jnp.exp(a - m) + jnp.exp(b - m))
    out_ref[...] = logits - lse


def kernel(cfeats, cedge_feats, ctypes, tfeats, tedge_feats, combined_feats,
           cgraph_edge_index, tgraph_edge_index, params):
    del combined_feats
    op_emb = params['op_emb']
    c_w, c_b = params['c_W'], params['c_b']
    t_w, t_b = params['t_W'], params['t_b']

    c_wx = [w[:-16, :] for w in c_w]
    c_we = [w[-16:, :] for w in c_w]
    t_wx = [w[:-16, :] for w in t_w]
    t_we = [w[-16:, :] for w in t_w]
    c_b_stack = jnp.stack(c_b, axis=0)
    t_b_stack = jnp.stack(t_b, axis=0)

    gnn_in = ([cfeats, ctypes.astype(jnp.int32).reshape(1, N_NODE), op_emb,
               cgraph_edge_index.astype(jnp.int32),
               cedge_feats, tfeats,
               tgraph_edge_index.astype(jnp.int32), tedge_feats]
              + c_wx + c_we + [c_b_stack] + t_wx + t_we + [t_b_stack])

    c_emb, t_emb = pl.pallas_call(
        _gnn_kernel,
        out_shape=[jax.ShapeDtypeStruct((N_NODE, H), _F32),
                   jax.ShapeDtypeStruct((N_NODE, H), _F32)],
    )(*gnn_in)

    k1_f, r1_f, b1_f = params['gru1_f']
    k1_b, r1_b, b1_b = params['gru1_b']
    k2_f, r2_f, b2_f = params['gru2_f']
    k2_b, r2_b, b2_b = params['gru2_b']
    fw = params['final_W']
    fb = params['final_b']

    out_tc = pl.pallas_call(
        _rnn_kernel,
        out_shape=jax.ShapeDtypeStruct((T, C, 2), _F32),
        scratch_shapes=[pltpu.VMEM((T, C, 2 * RH), _F32),
                        pltpu.VMEM((T, 3 * RH), _F32),
                        pltpu.VMEM((T, 3 * RH), _F32)],
    )(c_emb, t_emb,
      k1_f[:H, :], k1_f[H:, :], r1_f, b1_f,
      k1_b[:H, :], k1_b[H:, :], r1_b, b1_b,
      k2_f, r2_f, b2_f,
      k2_b, r2_b, b2_b,
      fw[:RH, :], fw[RH:, :], fb.reshape(1, 2))

    return jnp.swapaxes(out_tc, 0, 1)


# speedup vs baseline: 1.5307x; 1.1313x over previous
import jax
import jax.numpy as jnp
from jax.experimental import pallas as pl
from jax.experimental.pallas import tpu as pltpu

N_NODE = 128
H = 256
RH = 128
T = 128
C = 128

_F32 = jnp.float32


_BF16 = jnp.bfloat16


def _dot(a, b):
    return jax.lax.dot_general(a, b, (((1,), (0,)), ((), ())),
                               preferred_element_type=_F32)


def _dotb(a, b):
    return jax.lax.dot_general(a.astype(_BF16), b, (((1,), (0,)), ((), ())),
                               preferred_element_type=_F32)


def _gnn_kernel(cfeats_ref, ctypes_ref, op_emb_ref, c_ei_ref, c_ef_ref,
                tfeats_ref, t_ei_ref, t_ef_ref,
                *rest):
    c_wx = rest[0:6]
    c_we = rest[6:12]
    c_b_ref = rest[12]
    t_wx = rest[13:17]
    t_we = rest[17:21]
    t_b_ref = rest[21]
    c_out_ref, t_out_ref = rest[22], rest[23]

    def run_graph(x0, ei_ref, ef_ref, wx_refs, we_refs, b_ref, acts, resids):
        n_e = ei_ref.shape[1]
        src = ei_ref[0:1, :]
        dst = ei_ref[1:2, :]
        e_iota = jax.lax.broadcasted_iota(jnp.int32, (N_NODE, n_e), 0)
        src1h = (e_iota == src).astype(_F32)
        dst1h = (e_iota == dst).astype(_F32)
        adj = jax.lax.dot_general(dst1h, src1h, (((1,), (1,)), ((), ())),
                                  preferred_element_type=_F32)
        eagg = _dot(dst1h, ef_ref[...])
        ones = jnp.ones((N_NODE, 1), _F32)
        in_deg = _dot(adj, ones)
        out_deg = jax.lax.dot_general(adj, ones, (((0,), (0,)), ((), ())),
                                      preferred_element_type=_F32)
        ood = jax.lax.rsqrt(jnp.maximum(out_deg, 1.0))
        ind = jax.lax.rsqrt(jnp.maximum(in_deg, 1.0))

        x = x0
        for i in range(len(wx_refs)):
            feat = x * ood
            agg = _dot(adj, feat)
            rst = _dot(agg, wx_refs[i][...]) + _dot(eagg, we_refs[i][...])
            rst = rst * ind + b_ref[i:i + 1, :]
            if acts[i]:
                rst = jax.nn.sigmoid(rst)
            if resids[i]:
                rst = feat + rst
            x = rst
        return x

    n_ops = op_emb_ref.shape[0]
    t_iota = jax.lax.broadcasted_iota(jnp.int32, (n_ops, N_NODE), 0)
    op1h_t = (t_iota == ctypes_ref[0:1, :]).astype(_F32)
    op = jax.lax.dot_general(op1h_t, op_emb_ref[...],
                             (((0,), (0,)), ((), ())),
                             preferred_element_type=_F32)
    x0c = jnp.concatenate([cfeats_ref[...], op], axis=1)

    c_emb = run_graph(x0c, c_ei_ref, c_ef_ref, c_wx, c_we, c_b_ref,
                      acts=[True, True, True, True, True, False],
                      resids=[False, True, True, True, True, False])
    c_out_ref[...] = c_emb

    t_emb = run_graph(tfeats_ref[...], t_ei_ref, t_ef_ref, t_wx, t_we,
                      t_b_ref,
                      acts=[True, True, True, False],
                      resids=[False, True, True, False])
    t_out_ref[...] = t_emb


def _gru_step(h, gx, gh):
    z = jax.nn.sigmoid(gx[:, 0:RH] + gh[:, 0:RH])
    r = jax.nn.sigmoid(gx[:, RH:2 * RH] + gh[:, RH:2 * RH])
    hc = jnp.tanh(gx[:, 2 * RH:] + r * gh[:, 2 * RH:])
    return z * h + (1.0 - z) * hc


def _rnn_kernel(c_emb_ref, t_emb_ref,
                kc_f_ref, kt_f_ref, r1_f_ref, b1_f_ref,
                kc_b_ref, kt_b_ref, r1_b_ref, b1_b_ref,
                k2_f_ref, r2_f_ref, b2_f_ref,
                k2_b_ref, r2_b_ref, b2_b_ref,
                wf_ref, wb_ref, fb_ref,
                out_ref, y1_ref, gxt_f_ref, gxt_b_ref):
    c_emb = c_emb_ref[...]
    t_emb = t_emb_ref[...]
    gxc_f = (_dot(c_emb, kc_f_ref[...])
             + b1_f_ref[0:1, :] + b1_f_ref[1:2, :])
    gxt_f_ref[...] = _dot(t_emb, kt_f_ref[...])
    gxc_b = (_dot(c_emb, kc_b_ref[...])
             + b1_b_ref[0:1, :] + b1_b_ref[1:2, :])
    gxt_b_ref[...] = _dot(t_emb, kt_b_ref[...])

    cb = c_emb.shape[0]
    r1_f = r1_f_ref[...].astype(_BF16)
    r1_b = r1_b_ref[...].astype(_BF16)

    def loop1(k, carry):
        h_f, h_b = carry
        tb = T - 1 - k
        gx_f = gxc_f + gxt_f_ref[pl.ds(k, 1), :]
        gx_b = gxc_b + gxt_b_ref[pl.ds(tb, 1), :]
        gh_f = _dotb(h_f, r1_f)
        gh_b = _dotb(h_b, r1_b)
        h_f = _gru_step(h_f, gx_f, gh_f)
        h_b = _gru_step(h_b, gx_b, gh_b)
        y1_ref[k, :, 0:RH] = h_f
        y1_ref[tb, :, RH:2 * RH] = h_b
        return h_f, h_b

    h0 = jnp.zeros((cb, RH), _F32)
    jax.lax.fori_loop(0, T, loop1, (h0, h0), unroll=2)

    k2_f = k2_f_ref[...].astype(_BF16)
    r2_f = r2_f_ref[...].astype(_BF16)
    k2_b = k2_b_ref[...].astype(_BF16)
    r2_b = r2_b_ref[...].astype(_BF16)
    b2f = b2_f_ref[0:1, :] + b2_f_ref[1:2, :]
    b2b = b2_b_ref[0:1, :] + b2_b_ref[1:2, :]
    wf = wf_ref[...]
    wb = wb_ref[...]

    out_ref[...] = jnp.broadcast_to(fb_ref[...].reshape(1, 1, 2), (T, cb, 2))

    def loop2(k, carry):
        h_f, h_b = carry
        tb = T - 1 - k
        x_f = y1_ref[k]
        x_b = y1_ref[tb]
        gx_f = _dotb(x_f, k2_f) + b2f
        gx_b = _dotb(x_b, k2_b) + b2b
        gh_f = _dotb(h_f, r2_f)
        gh_b = _dotb(h_b, r2_b)
        h_f = _gru_step(h_f, gx_f, gh_f)
        h_b = _gru_step(h_b, gx_b, gh_b)
        out_ref[k] = out_ref[k] + _dot(h_f, wf)
        out_ref[tb] = out_ref[tb] + _dot(h_b, wb)
        return h_f, h_b

    jax.lax.fori_loop(0, T, loop2, (h0, h0), unroll=2)

    logits = out_ref[...]
    a = logits[:, :, 0:1]
    b = logits[:, :, 1:2]
    m = jnp.maximum(a, b)
    lse = m + jnp.log(jnp.exp(a - m) + jnp.exp(b - m))
    out_ref[...] = logits - lse


def kernel(cfeats, cedge_feats, ctypes, tfeats, tedge_feats, combined_feats,
           cgraph_edge_index, tgraph_edge_index, params):
    del combined_feats
    op_emb = params['op_emb']
    c_w, c_b = params['c_W'], params['c_b']
    t_w, t_b = params['t_W'], params['t_b']

    c_wx = [w[:-16, :] for w in c_w]
    c_we = [w[-16:, :] for w in c_w]
    t_wx = [w[:-16, :] for w in t_w]
    t_we = [w[-16:, :] for w in t_w]
    c_b_stack = jnp.stack(c_b, axis=0)
    t_b_stack = jnp.stack(t_b, axis=0)

    gnn_in = ([cfeats, ctypes.astype(jnp.int32).reshape(1, N_NODE), op_emb,
               cgraph_edge_index.astype(jnp.int32),
               cedge_feats, tfeats,
               tgraph_edge_index.astype(jnp.int32), tedge_feats]
              + c_wx + c_we + [c_b_stack] + t_wx + t_we + [t_b_stack])

    c_emb, t_emb = pl.pallas_call(
        _gnn_kernel,
        out_shape=[jax.ShapeDtypeStruct((N_NODE, H), _F32),
                   jax.ShapeDtypeStruct((N_NODE, H), _F32)],
    )(*gnn_in)

    k1_f, r1_f, b1_f = params['gru1_f']
    k1_b, r1_b, b1_b = params['gru1_b']
    k2_f, r2_f, b2_f = params['gru2_f']
    k2_b, r2_b, b2_b = params['gru2_b']
    fw = params['final_W']
    fb = params['final_b']

    out_tc = pl.pallas_call(
        _rnn_kernel,
        out_shape=jax.ShapeDtypeStruct((T, C, 2), _F32),
        scratch_shapes=[pltpu.VMEM((T, C, 2 * RH), _F32),
                        pltpu.VMEM((T, 3 * RH), _F32),
                        pltpu.VMEM((T, 3 * RH), _F32)],
    )(c_emb, t_emb,
      k1_f[:H, :], k1_f[H:, :], r1_f, b1_f,
      k1_b[:H, :], k1_b[H:, :], r1_b, b1_b,
      k2_f, r2_f, b2_f,
      k2_b, r2_b, b2_b,
      fw[:RH, :], fw[RH:, :], fb.reshape(1, 2))

    return jnp.swapaxes(out_tc, 0, 1)


# speedup vs baseline: 1.6691x; 1.0904x over previous
import jax
import jax.numpy as jnp
from jax.experimental import pallas as pl
from jax.experimental.pallas import tpu as pltpu

N_NODE = 128
H = 256
RH = 128
T = 128
C = 128

_F32 = jnp.float32


_BF16 = jnp.bfloat16


def _dot(a, b):
    return jax.lax.dot_general(a, b, (((1,), (0,)), ((), ())),
                               preferred_element_type=_F32)


def _dotb(a, b):
    return jax.lax.dot_general(a.astype(_BF16), b, (((1,), (0,)), ((), ())),
                               preferred_element_type=_F32)


def _gnn_kernel(cfeats_ref, ctypes_ref, op_emb_ref, c_ei_ref, c_ef_ref,
                tfeats_ref, t_ei_ref, t_ef_ref,
                *rest):
    c_wx = rest[0:6]
    c_we = rest[6:12]
    c_b_ref = rest[12]
    t_wx = rest[13:17]
    t_we = rest[17:21]
    t_b_ref = rest[21]
    c_out_ref, t_out_ref = rest[22], rest[23]

    def run_graph(x0, ei_ref, ef_ref, wx_refs, we_refs, b_ref, acts, resids):
        n_e = ei_ref.shape[1]
        src = ei_ref[0:1, :]
        dst = ei_ref[1:2, :]
        e_iota = jax.lax.broadcasted_iota(jnp.int32, (N_NODE, n_e), 0)
        src1h = (e_iota == src).astype(_F32)
        dst1h = (e_iota == dst).astype(_F32)
        adj = jax.lax.dot_general(dst1h, src1h, (((1,), (1,)), ((), ())),
                                  preferred_element_type=_F32)
        eagg = _dot(dst1h, ef_ref[...])
        ones = jnp.ones((N_NODE, 1), _F32)
        in_deg = _dot(adj, ones)
        out_deg = jax.lax.dot_general(adj, ones, (((0,), (0,)), ((), ())),
                                      preferred_element_type=_F32)
        ood = jax.lax.rsqrt(jnp.maximum(out_deg, 1.0))
        ind = jax.lax.rsqrt(jnp.maximum(in_deg, 1.0))

        x = x0
        for i in range(len(wx_refs)):
            feat = x * ood
            agg = _dot(adj, feat)
            rst = _dot(agg, wx_refs[i][...]) + _dot(eagg, we_refs[i][...])
            rst = rst * ind + b_ref[i:i + 1, :]
            if acts[i]:
                rst = jax.nn.sigmoid(rst)
            if resids[i]:
                rst = feat + rst
            x = rst
        return x

    n_ops = op_emb_ref.shape[0]
    t_iota = jax.lax.broadcasted_iota(jnp.int32, (n_ops, N_NODE), 0)
    op1h_t = (t_iota == ctypes_ref[0:1, :]).astype(_F32)
    op = jax.lax.dot_general(op1h_t, op_emb_ref[...],
                             (((0,), (0,)), ((), ())),
                             preferred_element_type=_F32)
    x0c = jnp.concatenate([cfeats_ref[...], op], axis=1)

    c_emb = run_graph(x0c, c_ei_ref, c_ef_ref, c_wx, c_we, c_b_ref,
                      acts=[True, True, True, True, True, False],
                      resids=[False, True, True, True, True, False])
    c_out_ref[...] = c_emb

    t_emb = run_graph(tfeats_ref[...], t_ei_ref, t_ef_ref, t_wx, t_we,
                      t_b_ref,
                      acts=[True, True, True, False],
                      resids=[False, True, True, False])
    t_out_ref[...] = t_emb


def _gru_step(h, gx, gh):
    z = jax.nn.sigmoid(gx[:, 0:RH] + gh[:, 0:RH])
    r = jax.nn.sigmoid(gx[:, RH:2 * RH] + gh[:, RH:2 * RH])
    hc = jnp.tanh(gx[:, 2 * RH:] + r * gh[:, 2 * RH:])
    return z * h + (1.0 - z) * hc


def _rnn_kernel(c_emb_ref, t_emb_ref,
                kc_f_ref, kt_f_ref, r1_f_ref, b1_f_ref,
                kc_b_ref, kt_b_ref, r1_b_ref, b1_b_ref,
                k2_f_ref, r2_f_ref, b2_f_ref,
                k2_b_ref, r2_b_ref, b2_b_ref,
                wf_ref, wb_ref, fb_ref,
                out_ref, y1_ref, gxt_f_ref, gxt_b_ref):
    c_emb = c_emb_ref[...]
    t_emb = t_emb_ref[...]
    gxc_f = (_dot(c_emb, kc_f_ref[...])
             + b1_f_ref[0:1, :] + b1_f_ref[1:2, :])
    gxt_f_ref[...] = _dot(t_emb, kt_f_ref[...])
    gxc_b = (_dot(c_emb, kc_b_ref[...])
             + b1_b_ref[0:1, :] + b1_b_ref[1:2, :])
    gxt_b_ref[...] = _dot(t_emb, kt_b_ref[...])

    cb = c_emb.shape[0]
    r1_f = r1_f_ref[...].astype(_BF16)
    r1_b = r1_b_ref[...].astype(_BF16)

    def loop1(k, carry):
        h_f, h_b = carry
        tb = T - 1 - k
        gx_f = gxc_f + gxt_f_ref[pl.ds(k, 1), :]
        gx_b = gxc_b + gxt_b_ref[pl.ds(tb, 1), :]
        gh_f = _dotb(h_f, r1_f)
        gh_b = _dotb(h_b, r1_b)
        h_f = _gru_step(h_f, gx_f, gh_f)
        h_b = _gru_step(h_b, gx_b, gh_b)
        y1_ref[k, :, 0:RH] = h_f
        y1_ref[tb, :, RH:2 * RH] = h_b
        return h_f, h_b

    h0 = jnp.zeros((cb, RH), _F32)
    jax.lax.fori_loop(0, T, loop1, (h0, h0), unroll=4)

    k2_f = k2_f_ref[...].astype(_BF16)
    r2_f = r2_f_ref[...].astype(_BF16)
    k2_b = k2_b_ref[...].astype(_BF16)
    r2_b = r2_b_ref[...].astype(_BF16)
    b2f = b2_f_ref[0:1, :] + b2_f_ref[1:2, :]
    b2b = b2_b_ref[0:1, :] + b2_b_ref[1:2, :]
    wf = wf_ref[...]
    wb = wb_ref[...]

    out_ref[...] = jnp.broadcast_to(fb_ref[...].reshape(1, 1, 2), (T, cb, 2))

    def loop2(k, carry):
        h_f, h_b = carry
        tb = T - 1 - k
        x_f = y1_ref[k]
        x_b = y1_ref[tb]
        gx_f = _dotb(x_f, k2_f) + b2f
        gx_b = _dotb(x_b, k2_b) + b2b
        gh_f = _dotb(h_f, r2_f)
        gh_b = _dotb(h_b, r2_b)
        h_f = _gru_step(h_f, gx_f, gh_f)
        h_b = _gru_step(h_b, gx_b, gh_b)
        out_ref[k] = out_ref[k] + _dot(h_f, wf)
        out_ref[tb] = out_ref[tb] + _dot(h_b, wb)
        return h_f, h_b

    jax.lax.fori_loop(0, T, loop2, (h0, h0), unroll=4)

    logits = out_ref[...]
    a = logits[:, :, 0:1]
    b = logits[:, :, 1:2]
    m = jnp.maximum(a, b)
    lse = m + jnp.log(jnp.exp(a - m) + jnp.exp(b - m))
    out_ref[...] = logits - lse


def kernel(cfeats, cedge_feats, ctypes, tfeats, tedge_feats, combined_feats,
           cgraph_edge_index, tgraph_edge_index, params):
    del combined_feats
    op_emb = params['op_emb']
    c_w, c_b = params['c_W'], params['c_b']
    t_w, t_b = params['t_W'], params['t_b']

    c_wx = [w[:-16, :] for w in c_w]
    c_we = [w[-16:, :] for w in c_w]
    t_wx = [w[:-16, :] for w in t_w]
    t_we = [w[-16:, :] for w in t_w]
    c_b_stack = jnp.stack(c_b, axis=0)
    t_b_stack = jnp.stack(t_b, axis=0)

    gnn_in = ([cfeats, ctypes.astype(jnp.int32).reshape(1, N_NODE), op_emb,
               cgraph_edge_index.astype(jnp.int32),
               cedge_feats, tfeats,
               tgraph_edge_index.astype(jnp.int32), tedge_feats]
              + c_wx + c_we + [c_b_stack] + t_wx + t_we + [t_b_stack])

    c_emb, t_emb = pl.pallas_call(
        _gnn_kernel,
        out_shape=[jax.ShapeDtypeStruct((N_NODE, H), _F32),
                   jax.ShapeDtypeStruct((N_NODE, H), _F32)],
    )(*gnn_in)

    k1_f, r1_f, b1_f = params['gru1_f']
    k1_b, r1_b, b1_b = params['gru1_b']
    k2_f, r2_f, b2_f = params['gru2_f']
    k2_b, r2_b, b2_b = params['gru2_b']
    fw = params['final_W']
    fb = params['final_b']

    out_tc = pl.pallas_call(
        _rnn_kernel,
        out_shape=jax.ShapeDtypeStruct((T, C, 2), _F32),
        scratch_shapes=[pltpu.VMEM((T, C, 2 * RH), _F32),
                        pltpu.VMEM((T, 3 * RH), _F32),
                        pltpu.VMEM((T, 3 * RH), _F32)],
    )(c_emb, t_emb,
      k1_f[:H, :], k1_f[H:, :], r1_f, b1_f,
      k1_b[:H, :], k1_b[H:, :], r1_b, b1_b,
      k2_f, r2_f, b2_f,
      k2_b, r2_b, b2_b,
      fw[:RH, :], fw[RH:, :], fb.reshape(1, 2))

    return jnp.swapaxes(out_tc, 0, 1)


# speedup vs baseline: 1.7622x; 1.0558x over previous
import jax
import jax.numpy as jnp
from jax.experimental import pallas as pl
from jax.experimental.pallas import tpu as pltpu

N_NODE = 128
H = 256
RH = 128
T = 128
C = 128

_F32 = jnp.float32


_BF16 = jnp.bfloat16


def _dot(a, b):
    return jax.lax.dot_general(a, b, (((1,), (0,)), ((), ())),
                               preferred_element_type=_F32)


def _dotb(a, b):
    return jax.lax.dot_general(a.astype(_BF16), b, (((1,), (0,)), ((), ())),
                               preferred_element_type=_F32)


def _gnn_kernel(cfeats_ref, ctypes_ref, op_emb_ref, c_ei_ref, c_ef_ref,
                tfeats_ref, t_ei_ref, t_ef_ref,
                *rest):
    c_wx = rest[0:6]
    c_we = rest[6:12]
    c_b_ref = rest[12]
    t_wx = rest[13:17]
    t_we = rest[17:21]
    t_b_ref = rest[21]
    c_out_ref, t_out_ref = rest[22], rest[23]

    def run_graph(x0, ei_ref, ef_ref, wx_refs, we_refs, b_ref, acts, resids):
        n_e = ei_ref.shape[1]
        src = ei_ref[0:1, :]
        dst = ei_ref[1:2, :]
        e_iota = jax.lax.broadcasted_iota(jnp.int32, (N_NODE, n_e), 0)
        src1h = (e_iota == src).astype(_F32)
        dst1h = (e_iota == dst).astype(_F32)
        adj = jax.lax.dot_general(dst1h, src1h, (((1,), (1,)), ((), ())),
                                  preferred_element_type=_F32)
        eagg = _dot(dst1h, ef_ref[...])
        ones = jnp.ones((N_NODE, 1), _F32)
        in_deg = _dot(adj, ones)
        out_deg = jax.lax.dot_general(adj, ones, (((0,), (0,)), ((), ())),
                                      preferred_element_type=_F32)
        ood = jax.lax.rsqrt(jnp.maximum(out_deg, 1.0))
        ind = jax.lax.rsqrt(jnp.maximum(in_deg, 1.0))

        x = x0
        for i in range(len(wx_refs)):
            feat = x * ood
            agg = _dot(adj, feat)
            rst = _dot(agg, wx_refs[i][...]) + _dot(eagg, we_refs[i][...])
            rst = rst * ind + b_ref[i:i + 1, :]
            if acts[i]:
                rst = jax.nn.sigmoid(rst)
            if resids[i]:
                rst = feat + rst
            x = rst
        return x

    n_ops = op_emb_ref.shape[0]
    t_iota = jax.lax.broadcasted_iota(jnp.int32, (n_ops, N_NODE), 0)
    op1h_t = (t_iota == ctypes_ref[0:1, :]).astype(_F32)
    op = jax.lax.dot_general(op1h_t, op_emb_ref[...],
                             (((0,), (0,)), ((), ())),
                             preferred_element_type=_F32)
    x0c = jnp.concatenate([cfeats_ref[...], op], axis=1)

    c_emb = run_graph(x0c, c_ei_ref, c_ef_ref, c_wx, c_we, c_b_ref,
                      acts=[True, True, True, True, True, False],
                      resids=[False, True, True, True, True, False])
    c_out_ref[...] = c_emb

    t_emb = run_graph(tfeats_ref[...], t_ei_ref, t_ef_ref, t_wx, t_we,
                      t_b_ref,
                      acts=[True, True, True, False],
                      resids=[False, True, True, False])
    t_out_ref[...] = t_emb


def _gru_step(h, gx, gh):
    z = jax.nn.sigmoid(gx[:, 0:RH] + gh[:, 0:RH])
    r = jax.nn.sigmoid(gx[:, RH:2 * RH] + gh[:, RH:2 * RH])
    hc = jnp.tanh(gx[:, 2 * RH:] + r * gh[:, 2 * RH:])
    return z * h + (1.0 - z) * hc


def _rnn_kernel(c_emb_ref, t_emb_ref,
                kc_f_ref, kt_f_ref, r1_f_ref, b1_f_ref,
                kc_b_ref, kt_b_ref, r1_b_ref, b1_b_ref,
                k2_f_ref, r2_f_ref, b2_f_ref,
                k2_b_ref, r2_b_ref, b2_b_ref,
                wf_ref, wb_ref, fb_ref,
                out_ref, y1_ref, gxt_f_ref, gxt_b_ref):
    c_emb = c_emb_ref[...]
    t_emb = t_emb_ref[...]
    gxc_f = (_dot(c_emb, kc_f_ref[...])
             + b1_f_ref[0:1, :] + b1_f_ref[1:2, :])
    gxt_f_ref[...] = _dot(t_emb, kt_f_ref[...])
    gxc_b = (_dot(c_emb, kc_b_ref[...])
             + b1_b_ref[0:1, :] + b1_b_ref[1:2, :])
    gxt_b_ref[...] = _dot(t_emb, kt_b_ref[...])

    cb = c_emb.shape[0]
    r1_f = r1_f_ref[...].astype(_BF16)
    r1_b = r1_b_ref[...].astype(_BF16)

    def loop1(k, carry):
        h_f, h_b = carry
        tb = T - 1 - k
        gx_f = gxc_f + gxt_f_ref[pl.ds(k, 1), :]
        gx_b = gxc_b + gxt_b_ref[pl.ds(tb, 1), :]
        gh_f = _dotb(h_f, r1_f)
        gh_b = _dotb(h_b, r1_b)
        h_f = _gru_step(h_f, gx_f, gh_f)
        h_b = _gru_step(h_b, gx_b, gh_b)
        y1_ref[k, :, 0:RH] = h_f
        y1_ref[tb, :, RH:2 * RH] = h_b
        return h_f, h_b

    h0 = jnp.zeros((cb, RH), _F32)
    jax.lax.fori_loop(0, T, loop1, (h0, h0), unroll=8)

    k2_f = k2_f_ref[...].astype(_BF16)
    r2_f = r2_f_ref[...].astype(_BF16)
    k2_b = k2_b_ref[...].astype(_BF16)
    r2_b = r2_b_ref[...].astype(_BF16)
    b2f = b2_f_ref[0:1, :] + b2_f_ref[1:2, :]
    b2b = b2_b_ref[0:1, :] + b2_b_ref[1:2, :]
    wf = wf_ref[...]
    wb = wb_ref[...]

    out_ref[...] = jnp.broadcast_to(fb_ref[...].reshape(1, 1, 2), (T, cb, 2))

    def loop2(k, carry):
        h_f, h_b = carry
        tb = T - 1 - k
        x_f = y1_ref[k]
        x_b = y1_ref[tb]
        gx_f = _dotb(x_f, k2_f) + b2f
        gx_b = _dotb(x_b, k2_b) + b2b
        gh_f = _dotb(h_f, r2_f)
        gh_b = _dotb(h_b, r2_b)
        h_f = _gru_step(h_f, gx_f, gh_f)
        h_b = _gru_step(h_b, gx_b, gh_b)
        out_ref[k] = out_ref[k] + _dot(h_f, wf)
        out_ref[tb] = out_ref[tb] + _dot(h_b, wb)
        return h_f, h_b

    jax.lax.fori_loop(0, T, loop2, (h0, h0), unroll=8)

    logits = out_ref[...]
    a = logits[:, :, 0:1]
    b = logits[:, :, 1:2]
    m = jnp.maximum(a, b)
    lse = m + jnp.log(jnp.exp(a - m) + jnp.exp(b - m))
    out_ref[...] = logits - lse


def kernel(cfeats, cedge_feats, ctypes, tfeats, tedge_feats, combined_feats,
           cgraph_edge_index, tgraph_edge_index, params):
    del combined_feats
    op_emb = params['op_emb']
    c_w, c_b = params['c_W'], params['c_b']
    t_w, t_b = params['t_W'], params['t_b']

    c_wx = [w[:-16, :] for w in c_w]
    c_we = [w[-16:, :] for w in c_w]
    t_wx = [w[:-16, :] for w in t_w]
    t_we = [w[-16:, :] for w in t_w]
    c_b_stack = jnp.stack(c_b, axis=0)
    t_b_stack = jnp.stack(t_b, axis=0)

    gnn_in = ([cfeats, ctypes.astype(jnp.int32).reshape(1, N_NODE), op_emb,
               cgraph_edge_index.astype(jnp.int32),
               cedge_feats, tfeats,
               tgraph_edge_index.astype(jnp.int32), tedge_feats]
              + c_wx + c_we + [c_b_stack] + t_wx + t_we + [t_b_stack])

    c_emb, t_emb = pl.pallas_call(
        _gnn_kernel,
        out_shape=[jax.ShapeDtypeStruct((N_NODE, H), _F32),
                   jax.ShapeDtypeStruct((N_NODE, H), _F32)],
    )(*gnn_in)

    k1_f, r1_f, b1_f = params['gru1_f']
    k1_b, r1_b, b1_b = params['gru1_b']
    k2_f, r2_f, b2_f = params['gru2_f']
    k2_b, r2_b, b2_b = params['gru2_b']
    fw = params['final_W']
    fb = params['final_b']

    out_tc = pl.pallas_call(
        _rnn_kernel,
        out_shape=jax.ShapeDtypeStruct((T, C, 2), _F32),
        scratch_shapes=[pltpu.VMEM((T, C, 2 * RH), _F32),
                        pltpu.VMEM((T, 3 * RH), _F32),
                        pltpu.VMEM((T, 3 * RH), _F32)],
    )(c_emb, t_emb,
      k1_f[:H, :], k1_f[H:, :], r1_f, b1_f,
      k1_b[:H, :], k1_b[H:, :], r1_b, b1_b,
      k2_f, r2_f, b2_f,
      k2_b, r2_b, b2_b,
      fw[:RH, :], fw[RH:, :], fb.reshape(1, 2))

    return jnp.swapaxes(out_tc, 0, 1)
